# Initial kernel scaffold; baseline (speedup 1.0000x reference)
import jax
import jax.numpy as jnp
from jax.experimental import pallas as pl


def kernel(x, se_fc_w, se_fc_b, offx_w, offx_b, offy_w, offy_b, sx_w, sx_b, sy_w, sy_b):
    raise NotImplementedError("write your pallas kernel here")


# speedup vs baseline: 2.2788x; 2.2788x over previous
import functools

import jax
import jax.numpy as jnp
from jax import lax
from jax.experimental import pallas as pl
from jax.experimental.pallas import tpu as pltpu
from jax.experimental.pallas import tpu_sc as plsc

B, C, W, H = 4, 96, 224, 224
K = 48
NPIX = W * H
PLANES = B * K
EPS = 1e-6
P = 0.5

CB = 8
NCB = C // CB


def _sums_body(x_ref, sums_ref):
    part = jnp.sum(x_ref[...], axis=2)
    sums_ref[...] = part[None]


def _sums(x2):
    return pl.pallas_call(
        _sums_body,
        grid=(NCB,),
        in_specs=[pl.BlockSpec((B, CB, NPIX), lambda c8: (0, c8, 0))],
        out_specs=pl.BlockSpec((1, B, CB), lambda c8: (c8, 0, 0)),
        out_shape=jax.ShapeDtypeStruct((NCB, B, CB), jnp.float32),
    )(x2)


def _se_topk_body(sums_ref, w_ref, b_ref, idx_ref):
    s = sums_ref[...]
    means = jnp.transpose(s, (1, 0, 2)).reshape(B, C) * (1.0 / NPIX)
    y = lax.dot_general(means, w_ref[...], (((1,), (1,)), ((), ())),
                        preferred_element_type=jnp.float32) + b_ref[...]
    y = jnp.where(y > 0, y, 0.01 * y)
    yj = y[:, :, None]
    yi = y[:, None, :]
    jlt = (lax.broadcasted_iota(jnp.int32, (C, C), 0)
           < lax.broadcasted_iota(jnp.int32, (C, C), 1))[None]
    cnt = jnp.where((yj > yi) | ((yj == yi) & jlt), 1, 0)
    rank = jnp.sum(cnt.astype(jnp.int32), axis=1)
    r_iota = lax.broadcasted_iota(jnp.int32, (B, K, C), 1)
    i_iota = lax.broadcasted_iota(jnp.int32, (B, K, C), 2)
    oh = rank[:, None, :] == r_iota
    idx_ref[...] = jnp.sum(jnp.where(oh, i_iota, 0), axis=2)


def _se_topk(sums, se_w, se_b):
    return pl.pallas_call(
        _se_topk_body,
        in_specs=[
            pl.BlockSpec((NCB, B, CB), lambda: (0, 0, 0)),
            pl.BlockSpec((C, C), lambda: (0, 0)),
            pl.BlockSpec((1, C), lambda: (0, 0)),
        ],
        out_specs=pl.BlockSpec((B, K), lambda: (0, 0)),
        out_shape=jax.ShapeDtypeStruct((B, K), jnp.int32),
    )(sums, se_w, se_b.reshape(1, C))


def _aux_body(idx_sm, par_sm, x_ref, sel_ref, ex_ref, ey_ref, sx_ref, sy_ref):
    r = pl.program_id(1)
    t = x_ref[...]
    sel_ref[...] = t
    zx = t * par_sm[0, r] + par_sm[1, r]
    zy = t * par_sm[2, r] + par_sm[3, r]
    ex_ref[...] = jax.nn.sigmoid(zx) * (W - 1.0)
    ey_ref[...] = jax.nn.sigmoid(zy) * (H - 1.0)
    sx_ref[...] = jnp.abs(t * par_sm[4, r] + par_sm[5, r])
    sy_ref[...] = jnp.abs(t * par_sm[6, r] + par_sm[7, r])


def _aux(x, idx, params):
    blk = pl.BlockSpec((1, 1, W, H), lambda b, r, i_sm, p_sm: (b, r, 0, 0))
    shp = jax.ShapeDtypeStruct((B, K, W, H), jnp.float32)
    return pl.pallas_call(
        _aux_body,
        grid_spec=pltpu.PrefetchScalarGridSpec(
            num_scalar_prefetch=2,
            grid=(B, K),
            in_specs=[pl.BlockSpec((1, 1, W, H),
                                   lambda b, r, i_sm, p_sm: (b, i_sm[b, r], 0, 0))],
            out_specs=[blk, blk, blk, blk, blk],
        ),
        out_shape=[shp, shp, shp, shp, shp],
    )(idx, params, x)


NC, NS, L = 2, 16, 16
NW = NC * NS
JOBS = PLANES // NW
CH = NPIX // 8


def _swap_body(sel_hbm, ex_hbm, ey_hbm, sx_hbm, sy_hbm, out_hbm,
               plane_v, exv, eyv, sxv, syv, ov):
    wid = lax.axis_index("s") * NC + lax.axis_index("c")

    def job(j, carry):
        p = wid * JOBS + j
        pltpu.sync_copy(sel_hbm.at[p], plane_v)

        def chunk(ci, carry2):
            c0 = ci * CH
            pltpu.sync_copy(ex_hbm.at[p, pl.ds(c0, CH)], exv)
            pltpu.sync_copy(ey_hbm.at[p, pl.ds(c0, CH)], eyv)
            pltpu.sync_copy(sx_hbm.at[p, pl.ds(c0, CH)], sxv)
            pltpu.sync_copy(sy_hbm.at[p, pl.ds(c0, CH)], syv)

            def vec(i, carry3):
                s = i * L
                t = plane_v[pl.ds(c0 + s, L)]
                px = exv[pl.ds(s, L)]
                py = eyv[pl.ds(s, L)]
                sx = sxv[pl.ds(s, L)]
                sy = syv[pl.ds(s, L)]
                x0 = jnp.minimum(px.astype(jnp.int32), W - 1)
                x1 = jnp.minimum(x0 + 1, W - 1)
                y0 = jnp.minimum(py.astype(jnp.int32), H - 1)
                y1 = jnp.minimum(y0 + 1, H - 1)
                sx2 = 2.0 * sx * sx + EPS
                sy2 = 2.0 * sy * sy + EPS
                dx0 = x0.astype(jnp.float32) - px
                dx1 = x1.astype(jnp.float32) - px
                dy0 = y0.astype(jnp.float32) - py
                dy1 = y1.astype(jnp.float32) - py
                wx0 = jnp.exp(-(dx0 * dx0) / sx2)
                wx1 = jnp.exp(-(dx1 * dx1) / sx2)
                wy0 = jnp.exp(-(dy0 * dy0) / sy2)
                wy1 = jnp.exp(-(dy1 * dy1) / sy2)
                ix0 = x0 * H
                ix1 = x1 * H
                v00 = plsc.load_gather(plane_v, [ix0 + y0])
                v01 = plsc.load_gather(plane_v, [ix0 + y1])
                v10 = plsc.load_gather(plane_v, [ix1 + y0])
                v11 = plsc.load_gather(plane_v, [ix1 + y1])
                w00 = wx0 * wy0
                w01 = wx0 * wy1
                w10 = wx1 * wy0
                w11 = wx1 * wy1
                norm = w00 + w01 + w10 + w11 + EPS
                smp = (w00 * v00 + w01 * v01 + w10 * v10 + w11 * v11) / norm
                ov[pl.ds(s, L)] = (1.0 - P) * t + P * smp
                return carry3

            lax.fori_loop(0, CH // L, vec, 0)
            pltpu.sync_copy(ov, out_hbm.at[p, pl.ds(c0, CH)])
            return carry2

        lax.fori_loop(0, NPIX // CH, chunk, 0)
        return carry

    lax.fori_loop(0, JOBS, job, 0)


def _swap(sel2, ex2, ey2, sx2, sy2):
    mesh = plsc.VectorSubcoreMesh(core_axis_name="c", subcore_axis_name="s")
    f = functools.partial(
        pl.kernel,
        mesh=mesh,
        compiler_params=pltpu.CompilerParams(needs_layout_passes=False),
        out_type=jax.ShapeDtypeStruct((PLANES, NPIX), jnp.float32),
        scratch_types=[
            pltpu.VMEM((NPIX,), jnp.float32),
            pltpu.VMEM((CH,), jnp.float32),
            pltpu.VMEM((CH,), jnp.float32),
            pltpu.VMEM((CH,), jnp.float32),
            pltpu.VMEM((CH,), jnp.float32),
            pltpu.VMEM((CH,), jnp.float32),
        ],
    )(_swap_body)
    return f(sel2, ex2, ey2, sx2, sy2)


@jax.jit
def kernel(x, se_fc_w, se_fc_b, offx_w, offx_b, offy_w, offy_b,
           sx_w, sx_b, sy_w, sy_b):
    x2 = x.reshape(B, C, NPIX)
    sums = _sums(x2)
    idx = _se_topk(sums, se_fc_w, se_fc_b)
    params = jnp.stack([offx_w, offx_b, offy_w, offy_b,
                        sx_w, sx_b, sy_w, sy_b])
    sel, exPx, exPy, sigmax, sigmay = _aux(x, idx, params)
    swap = _swap(sel.reshape(PLANES, NPIX),
                 exPx.reshape(PLANES, NPIX),
                 exPy.reshape(PLANES, NPIX),
                 sigmax.reshape(PLANES, NPIX),
                 sigmay.reshape(PLANES, NPIX))
    out = jnp.concatenate([x, swap.reshape(B, K, W, H)], axis=1)
    return (out, exPx, exPy, sigmax, sigmay)


# speedup vs baseline: 2.8329x; 1.2431x over previous
import functools

import jax
import jax.numpy as jnp
from jax import lax
from jax.experimental import pallas as pl
from jax.experimental.pallas import tpu as pltpu
from jax.experimental.pallas import tpu_sc as plsc

B, C, W, H = 4, 96, 224, 224
K = 48
NPIX = W * H
PLANES = B * K
EPS = 1e-6
P = 0.5

CB = 8
NCB = C // CB


def _sums_body(x_ref, sums_ref):
    part = jnp.sum(x_ref[...], axis=2)
    sums_ref[...] = part[None]


def _sums(x2):
    return pl.pallas_call(
        _sums_body,
        grid=(NCB,),
        in_specs=[pl.BlockSpec((B, CB, NPIX), lambda c8: (0, c8, 0))],
        out_specs=pl.BlockSpec((1, B, CB), lambda c8: (c8, 0, 0)),
        out_shape=jax.ShapeDtypeStruct((NCB, B, CB), jnp.float32),
    )(x2)


def _se_topk_body(sums_ref, w_ref, b_ref, idx_ref):
    s = sums_ref[...]
    means = jnp.transpose(s, (1, 0, 2)).reshape(B, C) * (1.0 / NPIX)
    y = lax.dot_general(means, w_ref[...], (((1,), (1,)), ((), ())),
                        preferred_element_type=jnp.float32) + b_ref[...]
    y = jnp.where(y > 0, y, 0.01 * y)
    yj = y[:, :, None]
    yi = y[:, None, :]
    jlt = (lax.broadcasted_iota(jnp.int32, (C, C), 0)
           < lax.broadcasted_iota(jnp.int32, (C, C), 1))[None]
    cnt = jnp.where((yj > yi) | ((yj == yi) & jlt), 1, 0)
    rank = jnp.sum(cnt.astype(jnp.int32), axis=1)
    r_iota = lax.broadcasted_iota(jnp.int32, (B, K, C), 1)
    i_iota = lax.broadcasted_iota(jnp.int32, (B, K, C), 2)
    oh = rank[:, None, :] == r_iota
    idx_ref[...] = jnp.sum(jnp.where(oh, i_iota, 0), axis=2)


def _se_topk(sums, se_w, se_b):
    return pl.pallas_call(
        _se_topk_body,
        in_specs=[
            pl.BlockSpec((NCB, B, CB), lambda: (0, 0, 0)),
            pl.BlockSpec((C, C), lambda: (0, 0)),
            pl.BlockSpec((1, C), lambda: (0, 0)),
        ],
        out_specs=pl.BlockSpec((B, K), lambda: (0, 0)),
        out_shape=jax.ShapeDtypeStruct((B, K), jnp.int32),
    )(sums, se_w, se_b.reshape(1, C))


def _aux_body(idx_sm, par_sm, x_ref, sel_ref, ex_ref, ey_ref, sx_ref, sy_ref,
              w00_ref, w01_ref, w10_ref, w11_ref):
    r = pl.program_id(1)
    t = x_ref[...]
    sel_ref[...] = t
    zx = t * par_sm[0, r] + par_sm[1, r]
    zy = t * par_sm[2, r] + par_sm[3, r]
    px = jax.nn.sigmoid(zx) * (W - 1.0)
    py = jax.nn.sigmoid(zy) * (H - 1.0)
    sx = jnp.abs(t * par_sm[4, r] + par_sm[5, r])
    sy = jnp.abs(t * par_sm[6, r] + par_sm[7, r])
    ex_ref[...] = px
    ey_ref[...] = py
    sx_ref[...] = sx
    sy_ref[...] = sy
    x0 = jnp.minimum(px.astype(jnp.int32), W - 1)
    x1 = jnp.minimum(x0 + 1, W - 1)
    y0 = jnp.minimum(py.astype(jnp.int32), H - 1)
    y1 = jnp.minimum(y0 + 1, H - 1)
    rx = 1.0 / (2.0 * sx * sx + EPS)
    ry = 1.0 / (2.0 * sy * sy + EPS)
    dx0 = x0.astype(jnp.float32) - px
    dx1 = x1.astype(jnp.float32) - px
    dy0 = y0.astype(jnp.float32) - py
    dy1 = y1.astype(jnp.float32) - py
    ax0 = dx0 * dx0 * rx
    ax1 = dx1 * dx1 * rx
    ay0 = dy0 * dy0 * ry
    ay1 = dy1 * dy1 * ry
    w00 = jnp.exp(-(ax0 + ay0))
    w01 = jnp.exp(-(ax0 + ay1))
    w10 = jnp.exp(-(ax1 + ay0))
    w11 = jnp.exp(-(ax1 + ay1))
    s = P / (w00 + w01 + w10 + w11 + EPS)
    w00_ref[...] = w00 * s
    w01_ref[...] = w01 * s
    w10_ref[...] = w10 * s
    w11_ref[...] = w11 * s


def _aux(x, idx, params):
    blk = pl.BlockSpec((1, 1, W, H), lambda b, r, i_sm, p_sm: (b, r, 0, 0))
    shp = jax.ShapeDtypeStruct((B, K, W, H), jnp.float32)
    return pl.pallas_call(
        _aux_body,
        grid_spec=pltpu.PrefetchScalarGridSpec(
            num_scalar_prefetch=2,
            grid=(B, K),
            in_specs=[pl.BlockSpec((1, 1, W, H),
                                   lambda b, r, i_sm, p_sm: (b, i_sm[b, r], 0, 0))],
            out_specs=[blk] * 9,
        ),
        out_shape=[shp] * 9,
    )(idx, params, x)


NC, NS, L = 2, 16, 16
NW = NC * NS
JOBS = PLANES // NW
CH = NPIX // 8


def _swap_body(sel_hbm, ex_hbm, ey_hbm, w00_hbm, w01_hbm, w10_hbm, w11_hbm,
               out_hbm, plane_v, exv, eyv, w00v, w01v, w10v, w11v, ov):
    wid = lax.axis_index("s") * NC + lax.axis_index("c")

    def job(j, carry):
        p = wid * JOBS + j
        pltpu.sync_copy(sel_hbm.at[p], plane_v)

        def chunk(ci, carry2):
            c0 = ci * CH
            pltpu.sync_copy(ex_hbm.at[p, pl.ds(c0, CH)], exv)
            pltpu.sync_copy(ey_hbm.at[p, pl.ds(c0, CH)], eyv)
            pltpu.sync_copy(w00_hbm.at[p, pl.ds(c0, CH)], w00v)
            pltpu.sync_copy(w01_hbm.at[p, pl.ds(c0, CH)], w01v)
            pltpu.sync_copy(w10_hbm.at[p, pl.ds(c0, CH)], w10v)
            pltpu.sync_copy(w11_hbm.at[p, pl.ds(c0, CH)], w11v)

            def vec(i, carry3):
                s = i * L
                t = plane_v[pl.ds(c0 + s, L)]
                px = exv[pl.ds(s, L)]
                py = eyv[pl.ds(s, L)]
                x0 = jnp.minimum(px.astype(jnp.int32), W - 1)
                x1 = jnp.minimum(x0 + 1, W - 1)
                y0 = jnp.minimum(py.astype(jnp.int32), H - 1)
                y1 = jnp.minimum(y0 + 1, H - 1)
                ix0 = x0 * H
                ix1 = x1 * H
                v00 = plsc.load_gather(plane_v, [ix0 + y0])
                v01 = plsc.load_gather(plane_v, [ix0 + y1])
                v10 = plsc.load_gather(plane_v, [ix1 + y0])
                v11 = plsc.load_gather(plane_v, [ix1 + y1])
                acc = (1.0 - P) * t
                acc = acc + w00v[pl.ds(s, L)] * v00
                acc = acc + w01v[pl.ds(s, L)] * v01
                acc = acc + w10v[pl.ds(s, L)] * v10
                acc = acc + w11v[pl.ds(s, L)] * v11
                ov[pl.ds(s, L)] = acc
                return carry3

            lax.fori_loop(0, CH // L, vec, 0)
            pltpu.sync_copy(ov, out_hbm.at[p, pl.ds(c0, CH)])
            return carry2

        lax.fori_loop(0, NPIX // CH, chunk, 0)
        return carry

    lax.fori_loop(0, JOBS, job, 0)


def _swap(sel2, ex2, ey2, w00, w01, w10, w11):
    mesh = plsc.VectorSubcoreMesh(core_axis_name="c", subcore_axis_name="s")
    f = functools.partial(
        pl.kernel,
        mesh=mesh,
        compiler_params=pltpu.CompilerParams(needs_layout_passes=False),
        out_type=jax.ShapeDtypeStruct((PLANES, NPIX), jnp.float32),
        scratch_types=[pltpu.VMEM((NPIX,), jnp.float32)]
        + [pltpu.VMEM((CH,), jnp.float32)] * 7,
    )(_swap_body)
    return f(sel2, ex2, ey2, w00, w01, w10, w11)


@jax.jit
def kernel(x, se_fc_w, se_fc_b, offx_w, offx_b, offy_w, offy_b,
           sx_w, sx_b, sy_w, sy_b):
    x2 = x.reshape(B, C, NPIX)
    sums = _sums(x2)
    idx = _se_topk(sums, se_fc_w, se_fc_b)
    params = jnp.stack([offx_w, offx_b, offy_w, offy_b,
                        sx_w, sx_b, sy_w, sy_b])
    (sel, exPx, exPy, sigmax, sigmay,
     w00, w01, w10, w11) = _aux(x, idx, params)
    swap = _swap(sel.reshape(PLANES, NPIX),
                 exPx.reshape(PLANES, NPIX),
                 exPy.reshape(PLANES, NPIX),
                 w00.reshape(PLANES, NPIX),
                 w01.reshape(PLANES, NPIX),
                 w10.reshape(PLANES, NPIX),
                 w11.reshape(PLANES, NPIX))
    out = jnp.concatenate([x, swap.reshape(B, K, W, H)], axis=1)
    return (out, exPx, exPy, sigmax, sigmay)


# speedup vs baseline: 5.3243x; 1.8795x over previous
import functools

import jax
import jax.numpy as jnp
from jax import lax
from jax.experimental import pallas as pl
from jax.experimental.pallas import tpu as pltpu
from jax.experimental.pallas import tpu_sc as plsc

B, C, W, H = 4, 96, 224, 224
K = 48
NPIX = W * H
PLANES = B * K
EPS = 1e-6
P = 0.5

CB = 8
NCB = C // CB


def _sums_body(x_ref, sums_ref):
    part = jnp.sum(x_ref[...], axis=(2, 3))
    sums_ref[...] = part[None]


def _sums(x):
    return pl.pallas_call(
        _sums_body,
        grid=(NCB,),
        in_specs=[pl.BlockSpec((B, CB, W, H), lambda c8: (0, c8, 0, 0))],
        out_specs=pl.BlockSpec((1, B, CB), lambda c8: (c8, 0, 0)),
        out_shape=jax.ShapeDtypeStruct((NCB, B, CB), jnp.float32),
    )(x)


def _se_topk_body(sums_ref, w_ref, b_ref, idx_ref):
    s = sums_ref[...]
    means = jnp.transpose(s, (1, 0, 2)).reshape(B, C) * (1.0 / NPIX)
    y = lax.dot_general(means, w_ref[...], (((1,), (1,)), ((), ())),
                        preferred_element_type=jnp.float32) + b_ref[...]
    y = jnp.where(y > 0, y, 0.01 * y)
    yj = y[:, :, None]
    yi = y[:, None, :]
    jlt = (lax.broadcasted_iota(jnp.int32, (C, C), 0)
           < lax.broadcasted_iota(jnp.int32, (C, C), 1))[None]
    cnt = jnp.where((yj > yi) | ((yj == yi) & jlt), 1, 0)
    rank = jnp.sum(cnt.astype(jnp.int32), axis=1)
    r_iota = lax.broadcasted_iota(jnp.int32, (B, K, C), 1)
    i_iota = lax.broadcasted_iota(jnp.int32, (B, K, C), 2)
    oh = rank[:, None, :] == r_iota
    idx_ref[...] = jnp.sum(jnp.where(oh, i_iota, 0), axis=2)


def _se_topk(sums, se_w, se_b):
    return pl.pallas_call(
        _se_topk_body,
        in_specs=[
            pl.BlockSpec((NCB, B, CB), lambda: (0, 0, 0)),
            pl.BlockSpec((C, C), lambda: (0, 0)),
            pl.BlockSpec((1, C), lambda: (0, 0)),
        ],
        out_specs=pl.BlockSpec((B, K), lambda: (0, 0)),
        out_shape=jax.ShapeDtypeStruct((B, K), jnp.int32),
    )(sums, se_w, se_b.reshape(1, C))


def _aux_body(idx_sm, par_sm, x_ref, sel_ref, ex_ref, ey_ref, sx_ref, sy_ref,
              w00_ref, w01_ref, w10_ref, w11_ref, iw_ref):
    r = pl.program_id(1)
    t = x_ref[...]
    sel_ref[...] = t
    zx = t * par_sm[0, r] + par_sm[1, r]
    zy = t * par_sm[2, r] + par_sm[3, r]
    px = jax.nn.sigmoid(zx) * (W - 1.0)
    py = jax.nn.sigmoid(zy) * (H - 1.0)
    sx = jnp.abs(t * par_sm[4, r] + par_sm[5, r])
    sy = jnp.abs(t * par_sm[6, r] + par_sm[7, r])
    ex_ref[...] = px
    ey_ref[...] = py
    sx_ref[...] = sx
    sy_ref[...] = sy
    x0 = jnp.minimum(px.astype(jnp.int32), W - 1)
    x1 = jnp.minimum(x0 + 1, W - 1)
    y0 = jnp.minimum(py.astype(jnp.int32), H - 1)
    y1 = jnp.minimum(y0 + 1, H - 1)
    iw_ref[...] = (y0 + x0 * 256 + (y1 - y0) * 65536 + (x1 - x0) * 131072)
    rx = 1.0 / (2.0 * sx * sx + EPS)
    ry = 1.0 / (2.0 * sy * sy + EPS)
    dx0 = x0.astype(jnp.float32) - px
    dx1 = x1.astype(jnp.float32) - px
    dy0 = y0.astype(jnp.float32) - py
    dy1 = y1.astype(jnp.float32) - py
    ax0 = dx0 * dx0 * rx
    ax1 = dx1 * dx1 * rx
    ay0 = dy0 * dy0 * ry
    ay1 = dy1 * dy1 * ry
    w00 = jnp.exp(-(ax0 + ay0))
    w01 = jnp.exp(-(ax0 + ay1))
    w10 = jnp.exp(-(ax1 + ay0))
    w11 = jnp.exp(-(ax1 + ay1))
    s = P / (w00 + w01 + w10 + w11 + EPS)
    w00_ref[...] = w00 * s
    w01_ref[...] = w01 * s
    w10_ref[...] = w10 * s
    w11_ref[...] = w11 * s


def _aux(x, idx, params):
    blk = pl.BlockSpec((1, 1, W, H), lambda b, r, i_sm, p_sm: (b, r, 0, 0))
    shp = jax.ShapeDtypeStruct((B, K, W, H), jnp.float32)
    ishp = jax.ShapeDtypeStruct((B, K, W, H), jnp.int32)
    return pl.pallas_call(
        _aux_body,
        grid_spec=pltpu.PrefetchScalarGridSpec(
            num_scalar_prefetch=2,
            grid=(B, K),
            in_specs=[pl.BlockSpec((1, 1, W, H),
                                   lambda b, r, i_sm, p_sm: (b, i_sm[b, r], 0, 0))],
            out_specs=[blk] * 10,
        ),
        out_shape=[shp] * 9 + [ishp],
    )(idx, params, x)


NC, NS, L = 2, 16, 16
NW = NC * NS
JOBS = PLANES // NW
RW = 16
CH = RW * H
NCHUNK = W // RW
VPR = H // L


def _swap_body(sel_hbm, iw_hbm, w00_hbm, w01_hbm, w10_hbm, w11_hbm,
               out_hbm, plane_v, bufs0, bufs1, ov0, ov1,
               sem_i0, sem_i1, sem_o0, sem_o1):
    wid = lax.axis_index("s") * NC + lax.axis_index("c")
    b = lax.shift_right_logical(wid, 3)

    def job(j, carry):
        p = wid * JOBS + j
        kk = p - b * K
        pltpu.sync_copy(sel_hbm.at[b, kk], plane_v)

        def in_copies(ci, bufs, sem):
            r0 = ci * RW
            srcs = (iw_hbm, w00_hbm, w01_hbm, w10_hbm, w11_hbm)
            return [pltpu.make_async_copy(s.at[b, kk, pl.ds(r0, RW)], v, sem)
                    for s, v in zip(srcs, bufs)]

        def start_in(ci, bufs, sem):
            for c in in_copies(ci, bufs, sem):
                c.start()

        def wait_in(ci, bufs, sem):
            for c in in_copies(ci, bufs, sem):
                c.wait()

        def out_copy(ci, ov, sem):
            return pltpu.make_async_copy(
                ov, out_hbm.at[b, kk, pl.ds(ci * RW, RW)], sem)

        def compute(ci, bufs, ov):
            iwv, w00v, w01v, w10v, w11v = bufs
            r0 = ci * RW

            def row(rr, carry3):
                for cc in range(VPR):
                    cs = cc * L
                    t = plane_v[r0 + rr, pl.ds(cs, L)]
                    w = iwv[rr, pl.ds(cs, L)]
                    y0 = w & 255
                    x0 = lax.shift_right_logical(w, 8) & 255
                    y1 = y0 + (lax.shift_right_logical(w, 16) & 1)
                    x1 = x0 + lax.shift_right_logical(w, 17)
                    v00 = plsc.load_gather(plane_v, [x0, y0])
                    v01 = plsc.load_gather(plane_v, [x0, y1])
                    v10 = plsc.load_gather(plane_v, [x1, y0])
                    v11 = plsc.load_gather(plane_v, [x1, y1])
                    acc = (1.0 - P) * t
                    acc = acc + w00v[rr, pl.ds(cs, L)] * v00
                    acc = acc + w01v[rr, pl.ds(cs, L)] * v01
                    acc = acc + w10v[rr, pl.ds(cs, L)] * v10
                    acc = acc + w11v[rr, pl.ds(cs, L)] * v11
                    ov[rr, pl.ds(cs, L)] = acc
                return carry3

            lax.fori_loop(0, RW, row, 0)

        start_in(0, bufs0, sem_i0)

        def half(h, carry2):
            ci0 = 2 * h
            ci1 = ci0 + 1
            start_in(ci1, bufs1, sem_i1)
            wait_in(ci0, bufs0, sem_i0)

            @pl.when(h > 0)
            def _():
                out_copy(ci0 - 2, ov0, sem_o0).wait()

            compute(ci0, bufs0, ov0)
            out_copy(ci0, ov0, sem_o0).start()

            @pl.when(h < NCHUNK // 2 - 1)
            def _():
                start_in(ci0 + 2, bufs0, sem_i0)

            wait_in(ci1, bufs1, sem_i1)

            @pl.when(h > 0)
            def _():
                out_copy(ci1 - 2, ov1, sem_o1).wait()

            compute(ci1, bufs1, ov1)
            out_copy(ci1, ov1, sem_o1).start()
            return carry2

        lax.fori_loop(0, NCHUNK // 2, half, 0)
        out_copy(NCHUNK - 2, ov0, sem_o0).wait()
        out_copy(NCHUNK - 1, ov1, sem_o1).wait()
        return carry

    lax.fori_loop(0, JOBS, job, 0)


def _swap(sel, iw, w00, w01, w10, w11):
    mesh = plsc.VectorSubcoreMesh(core_axis_name="c", subcore_axis_name="s")
    buf = lambda dt: [pltpu.VMEM((RW, H), dt)]
    bufset = buf(jnp.int32) + buf(jnp.float32) * 4
    f = functools.partial(
        pl.kernel,
        mesh=mesh,
        compiler_params=pltpu.CompilerParams(needs_layout_passes=False),
        out_type=jax.ShapeDtypeStruct((B, K, W, H), jnp.float32),
        scratch_types=[
            pltpu.VMEM((W, H), jnp.float32),
            bufset, bufset,
            pltpu.VMEM((RW, H), jnp.float32),
            pltpu.VMEM((RW, H), jnp.float32),
            pltpu.SemaphoreType.DMA,
            pltpu.SemaphoreType.DMA,
            pltpu.SemaphoreType.DMA,
            pltpu.SemaphoreType.DMA,
        ],
    )(_swap_body)
    return f(sel, iw, w00, w01, w10, w11)


@jax.jit
def kernel(x, se_fc_w, se_fc_b, offx_w, offx_b, offy_w, offy_b,
           sx_w, sx_b, sy_w, sy_b):
    sums = _sums(x)
    idx = _se_topk(sums, se_fc_w, se_fc_b)
    params = jnp.stack([offx_w, offx_b, offy_w, offy_b,
                        sx_w, sx_b, sy_w, sy_b])
    (sel, exPx, exPy, sigmax, sigmay,
     w00, w01, w10, w11, iw) = _aux(x, idx, params)
    swap = _swap(sel, iw, w00, w01, w10, w11)
    out = jnp.concatenate([x, swap], axis=1)
    return (out, exPx, exPy, sigmax, sigmay)
